# baseline (speedup 1.0000x reference)
import functools

import jax
import jax.numpy as jnp
from jax import lax
from jax.experimental import pallas as pl
from jax.experimental.pallas import tpu as pltpu
from jax.experimental.pallas import tpu_sc as plsc

VOCAB = 100000
SEQ_LEN = 200
EMBED_DIM = 128
BATCH = 1024

NUM_CORES = 2
NUM_SUBCORES = 16
NUM_WORKERS = NUM_CORES * NUM_SUBCORES
BPW = BATCH // NUM_WORKERS
PAIRS = BPW // 2
IDX_PER_W = BPW * SEQ_LEN
LANES = 16
GROUPS = EMBED_DIM // LANES
SCALE = float(EMBED_DIM) ** 0.5
INV_SCALE = 1.0 / SCALE
NBUF = 3
CH = 112
NROWS = (112, 88)
NFULL = (7, 5)
NTAIL = (0, 8)


def _embed_kernel(idx_hbm, table_hbm, pos_hbm, out_hbm, idx_v, pos_v, buf,
                  gsem, osem):
    wid = lax.axis_index("s") * NUM_CORES + lax.axis_index("c")
    row0 = wid * IDX_PER_W

    pltpu.sync_copy(idx_hbm.at[pl.ds(row0, IDX_PER_W)],
                    idx_v.at[pl.ds(0, IDX_PER_W)])
    pltpu.sync_copy(pos_hbm, pos_v)

    @plsc.parallel_loop(0, SEQ_LEN)
    def _rescale(r):
        for g in range(GROUPS):
            sl = pl.ds(g * LANES, LANES)
            pos_v[r, sl] = pos_v[r, sl] * INV_SCALE

    def gdescs(p, half, sp):
        n = NROWS[half]
        ib = p * 2 * SEQ_LEN + half * CH
        base = sp * 2 * CH
        return (
            pltpu.make_async_copy(table_hbm.at[idx_v.at[pl.ds(ib, n)]],
                                  buf.at[pl.ds(base, n)], gsem),
            pltpu.make_async_copy(
                table_hbm.at[idx_v.at[pl.ds(ib + SEQ_LEN, n)]],
                buf.at[pl.ds(base + CH, n)], gsem),
        )

    def odescs(p, half, sp):
        n = NROWS[half]
        orow = row0 + p * 2 * SEQ_LEN + half * CH
        base = sp * 2 * CH
        return (
            pltpu.make_async_copy(buf.at[pl.ds(base, n)],
                                  out_hbm.at[pl.ds(orow, n)], osem),
            pltpu.make_async_copy(buf.at[pl.ds(base + CH, n)],
                                  out_hbm.at[pl.ds(orow + SEQ_LEN, n)], osem),
        )

    def step(p, half):
        c = 2 * p + half
        sp = lax.rem(c, NBUF)
        spn = lax.rem(c + 2, NBUF)
        base = sp * 2 * CH
        ib = p * 2 * SEQ_LEN + half * CH
        srow = half * CH

        for d in gdescs(p, half, sp):
            d.wait()

        def process16(off, nk):
            idxg_a = idx_v[pl.ds(ib + off, LANES)]
            idxg_b = idx_v[pl.ds(ib + SEQ_LEN + off, LANES)]
            m_a = jnp.where(idxg_a != 0, SCALE, 0.0).astype(jnp.float32)
            m_b = jnp.where(idxg_b != 0, SCALE, 0.0).astype(jnp.float32)
            for k in range(nk):
                ra = base + off + k
                rb = ra + CH
                s = srow + off + k
                mak = jnp.full((LANES,), m_a[k], jnp.float32)
                mbk = jnp.full((LANES,), m_b[k], jnp.float32)
                for g in range(GROUPS):
                    sl = pl.ds(g * LANES, LANES)
                    pv = pos_v[s, sl]
                    buf[ra, sl] = (buf[ra, sl] + pv) * mak
                    buf[rb, sl] = (buf[rb, sl] + pv) * mbk

        @plsc.parallel_loop(0, NFULL[half])
        def _sgrp(i):
            process16(i * LANES, LANES)

        if NTAIL[half]:
            process16(NFULL[half] * LANES, NTAIL[half])

        for d in odescs(p, half, sp):
            d.start()

        def drain_prev():
            pp, ph = (p - 1, 1) if half == 0 else (p, 0)
            for d in odescs(pp, ph, spn):
                d.wait()

        if half == 0:
            @pl.when(c >= 1)
            def _():
                drain_prev()
        else:
            drain_prev()

        @pl.when(p < PAIRS - 1)
        def _():
            for d in gdescs(p + 1, half, spn):
                d.start()

    for d in gdescs(0, 0, 0):
        d.start()
    for d in gdescs(0, 1, 1):
        d.start()

    def pair_body(p, _):
        step(p, 0)
        step(p, 1)
        return 0

    lax.fori_loop(0, PAIRS, pair_body, 0)
    for d in odescs(PAIRS - 1, 1, (2 * PAIRS - 1) % NBUF):
        d.wait()


@jax.jit
def _embed(idx_flat, token_table, position_table):
    mesh = plsc.VectorSubcoreMesh(core_axis_name="c", subcore_axis_name="s")
    kern = functools.partial(
        pl.kernel,
        out_type=jax.ShapeDtypeStruct((BATCH * SEQ_LEN, EMBED_DIM),
                                      jnp.float32),
        mesh=mesh,
        scratch_types=[
            pltpu.VMEM((IDX_PER_W + LANES,), jnp.int32),
            pltpu.VMEM((SEQ_LEN, EMBED_DIM), jnp.float32),
            pltpu.VMEM((NBUF * 2 * CH, EMBED_DIM), jnp.float32),
            pltpu.SemaphoreType.DMA,
            pltpu.SemaphoreType.DMA,
        ],
    )(_embed_kernel)
    return kern(idx_flat, token_table, position_table)


def kernel(inputs, token_table, position_table):
    idx_flat = inputs.astype(jnp.int32).reshape(-1)
    out = _embed(idx_flat, token_table, position_table)
    return out.reshape(BATCH, SEQ_LEN, EMBED_DIM)


# speedup vs baseline: 2.0373x; 2.0373x over previous
import functools

import jax
import jax.numpy as jnp
from jax import lax
from jax.experimental import pallas as pl
from jax.experimental.pallas import tpu as pltpu
from jax.experimental.pallas import tpu_sc as plsc

VOCAB = 100000
SEQ_LEN = 200
EMBED_DIM = 128
BATCH = 1024

NUM_CORES = 2
NUM_SUBCORES = 16
NUM_WORKERS = NUM_CORES * NUM_SUBCORES
BPW = BATCH // NUM_WORKERS
PAIRS = BPW // 2
IDX_PER_W = BPW * SEQ_LEN
LANES = 16
GROUPS = EMBED_DIM // LANES
SCALE = float(EMBED_DIM) ** 0.5
INV_SCALE = 1.0 / SCALE
NBUF = 3
CH = 112
NROWS = (112, 88)
NFULL = (7, 5)
NTAIL = (0, 8)


def _embed_kernel(idx_hbm, table_hbm, pos_hbm, out_hbm, idx_v, pos_v, buf,
                  gsem, osem):
    wid = lax.axis_index("s") * NUM_CORES + lax.axis_index("c")
    row0 = wid * IDX_PER_W

    pltpu.sync_copy(idx_hbm.at[pl.ds(row0, IDX_PER_W)],
                    idx_v.at[pl.ds(0, IDX_PER_W)])
    pltpu.sync_copy(pos_hbm, pos_v)

    @plsc.parallel_loop(0, SEQ_LEN)
    def _rescale(r):
        for g in range(GROUPS):
            sl = pl.ds(g * LANES, LANES)
            pos_v[r, sl] = pos_v[r, sl] * INV_SCALE

    def gdescs(p, half, sp):
        n = NROWS[half]
        ib = p * 2 * SEQ_LEN + half * CH
        base = sp * 2 * CH
        return (
            pltpu.make_async_copy(table_hbm.at[idx_v.at[pl.ds(ib, n)]],
                                  buf.at[pl.ds(base, n)], gsem),
            pltpu.make_async_copy(
                table_hbm.at[idx_v.at[pl.ds(ib + SEQ_LEN, n)]],
                buf.at[pl.ds(base + CH, n)], gsem),
        )

    def odescs(p, half, sp):
        n = NROWS[half]
        orow = row0 + p * 2 * SEQ_LEN + half * CH
        base = sp * 2 * CH
        return (
            pltpu.make_async_copy(buf.at[pl.ds(base, n)],
                                  out_hbm.at[pl.ds(orow, n)], osem),
            pltpu.make_async_copy(buf.at[pl.ds(base + CH, n)],
                                  out_hbm.at[pl.ds(orow + SEQ_LEN, n)], osem),
        )

    def step(p, half):
        c = 2 * p + half
        sp = lax.rem(c, NBUF)
        spn = lax.rem(c + 2, NBUF)
        base = sp * 2 * CH
        ib = p * 2 * SEQ_LEN + half * CH
        srow = half * CH

        for d in gdescs(p, half, sp):
            d.wait()

        def process16(off, nk):
            idxg_a = idx_v[pl.ds(ib + off, LANES)]
            idxg_b = idx_v[pl.ds(ib + SEQ_LEN + off, LANES)]
            m_a = jnp.where(idxg_a != 0, SCALE, 0.0).astype(jnp.float32)
            m_b = jnp.where(idxg_b != 0, SCALE, 0.0).astype(jnp.float32)
            for k in range(nk):
                ra = base + off + k
                rb = ra + CH
                s = srow + off + k
                mak = jnp.full((LANES,), m_a[k], jnp.float32)
                mbk = jnp.full((LANES,), m_b[k], jnp.float32)
                sls = [pl.ds(g * LANES, LANES) for g in range(GROUPS)]
                pvs = [pos_v[s, sl] for sl in sls]
                vas = [buf[ra, sl] for sl in sls]
                vbs = [buf[rb, sl] for sl in sls]
                for sl, pv, va in zip(sls, pvs, vas):
                    buf[ra, sl] = (va + pv) * mak
                for sl, pv, vb in zip(sls, pvs, vbs):
                    buf[rb, sl] = (vb + pv) * mbk

        @plsc.parallel_loop(0, NFULL[half])
        def _sgrp(i):
            process16(i * LANES, LANES)

        if NTAIL[half]:
            process16(NFULL[half] * LANES, NTAIL[half])

        for d in odescs(p, half, sp):
            d.start()

        def drain_prev():
            pp, ph = (p - 1, 1) if half == 0 else (p, 0)
            for d in odescs(pp, ph, spn):
                d.wait()

        if half == 0:
            @pl.when(c >= 1)
            def _():
                drain_prev()
        else:
            drain_prev()

        @pl.when(p < PAIRS - 1)
        def _():
            for d in gdescs(p + 1, half, spn):
                d.start()

    for d in gdescs(0, 0, 0):
        d.start()
    for d in gdescs(0, 1, 1):
        d.start()

    def pair_body(p, _):
        step(p, 0)
        step(p, 1)
        return 0

    lax.fori_loop(0, PAIRS, pair_body, 0)
    for d in odescs(PAIRS - 1, 1, (2 * PAIRS - 1) % NBUF):
        d.wait()


@jax.jit
def _embed(idx_flat, token_table, position_table):
    mesh = plsc.VectorSubcoreMesh(core_axis_name="c", subcore_axis_name="s")
    kern = functools.partial(
        pl.kernel,
        out_type=jax.ShapeDtypeStruct((BATCH * SEQ_LEN, EMBED_DIM),
                                      jnp.float32),
        mesh=mesh,
        scratch_types=[
            pltpu.VMEM((IDX_PER_W + LANES,), jnp.int32),
            pltpu.VMEM((SEQ_LEN, EMBED_DIM), jnp.float32),
            pltpu.VMEM((NBUF * 2 * CH, EMBED_DIM), jnp.float32),
            pltpu.SemaphoreType.DMA,
            pltpu.SemaphoreType.DMA,
        ],
    )(_embed_kernel)
    return kern(idx_flat, token_table, position_table)


def kernel(inputs, token_table, position_table):
    idx_flat = inputs.astype(jnp.int32).reshape(-1)
    out = _embed(idx_flat, token_table, position_table)
    return out.reshape(BATCH, SEQ_LEN, EMBED_DIM)


# speedup vs baseline: 2.0530x; 1.0077x over previous
import functools

import jax
import jax.numpy as jnp
from jax import lax
from jax.experimental import pallas as pl
from jax.experimental.pallas import tpu as pltpu
from jax.experimental.pallas import tpu_sc as plsc

VOCAB = 100000
SEQ_LEN = 200
EMBED_DIM = 128
BATCH = 1024

NUM_CORES = 2
NUM_SUBCORES = 16
NUM_WORKERS = NUM_CORES * NUM_SUBCORES
BPW = BATCH // NUM_WORKERS
PAIRS = BPW // 2
IDX_PER_W = BPW * SEQ_LEN
LANES = 16
GROUPS = EMBED_DIM // LANES
SCALE = float(EMBED_DIM) ** 0.5
INV_SCALE = 1.0 / SCALE
NBUF = 3
CH = 112
NROWS = (112, 88)
NFULL = (7, 5)
NTAIL = (0, 8)


def _embed_kernel(idx_hbm, table_hbm, pos_hbm, out_hbm, idx_v, pos_v, buf,
                  gsem, osem):
    wid = lax.axis_index("s") * NUM_CORES + lax.axis_index("c")
    row0 = wid * IDX_PER_W

    pltpu.sync_copy(idx_hbm.at[pl.ds(row0, IDX_PER_W)],
                    idx_v.at[pl.ds(0, IDX_PER_W)])
    pltpu.sync_copy(pos_hbm, pos_v)

    @plsc.parallel_loop(0, SEQ_LEN)
    def _rescale(r):
        for g in range(GROUPS):
            sl = pl.ds(g * LANES, LANES)
            pos_v[r, sl] = pos_v[r, sl] * INV_SCALE

    def gdescs(p, half, sp):
        n = NROWS[half]
        ib = p * 2 * SEQ_LEN + half * CH
        base = sp * 2 * CH
        return (
            pltpu.make_async_copy(table_hbm.at[idx_v.at[pl.ds(ib, n)]],
                                  buf.at[pl.ds(base, n)], gsem),
            pltpu.make_async_copy(
                table_hbm.at[idx_v.at[pl.ds(ib + SEQ_LEN, n)]],
                buf.at[pl.ds(base + CH, n)], gsem),
        )

    def odescs(p, half, sp):
        n = NROWS[half]
        orow = row0 + p * 2 * SEQ_LEN + half * CH
        base = sp * 2 * CH
        return (
            pltpu.make_async_copy(buf.at[pl.ds(base, n)],
                                  out_hbm.at[pl.ds(orow, n)], osem),
            pltpu.make_async_copy(buf.at[pl.ds(base + CH, n)],
                                  out_hbm.at[pl.ds(orow + SEQ_LEN, n)], osem),
        )

    def step(p, half):
        c = 2 * p + half
        sp = lax.rem(c, NBUF)
        spn = lax.rem(c + 2, NBUF)
        base = sp * 2 * CH
        ib = p * 2 * SEQ_LEN + half * CH
        srow = half * CH

        for d in gdescs(p, half, sp):
            d.wait()

        def process8(off):
            idxg_a = idx_v[pl.ds(ib + off, LANES)]
            idxg_b = idx_v[pl.ds(ib + SEQ_LEN + off, LANES)]
            m_a = jnp.where(idxg_a != 0, SCALE, 0.0).astype(jnp.float32)
            m_b = jnp.where(idxg_b != 0, SCALE, 0.0).astype(jnp.float32)
            for k in range(8):
                ra = base + off + k
                rb = ra + CH
                s = srow + off + k
                mak = jnp.full((LANES,), m_a[k], jnp.float32)
                mbk = jnp.full((LANES,), m_b[k], jnp.float32)
                sls = [pl.ds(g * LANES, LANES) for g in range(GROUPS)]
                pvs = [pos_v[s, sl] for sl in sls]
                vas = [buf[ra, sl] for sl in sls]
                vbs = [buf[rb, sl] for sl in sls]
                for sl, pv, va in zip(sls, pvs, vas):
                    buf[ra, sl] = (va + pv) * mak
                for sl, pv, vb in zip(sls, pvs, vbs):
                    buf[rb, sl] = (vb + pv) * mbk

        def process16(off, nk):
            idxg_a = idx_v[pl.ds(ib + off, LANES)]
            idxg_b = idx_v[pl.ds(ib + SEQ_LEN + off, LANES)]
            m_a = jnp.where(idxg_a != 0, SCALE, 0.0).astype(jnp.float32)
            m_b = jnp.where(idxg_b != 0, SCALE, 0.0).astype(jnp.float32)
            for k in range(nk):
                ra = base + off + k
                rb = ra + CH
                s = srow + off + k
                mak = jnp.full((LANES,), m_a[k], jnp.float32)
                mbk = jnp.full((LANES,), m_b[k], jnp.float32)
                sls = [pl.ds(g * LANES, LANES) for g in range(GROUPS)]
                pvs = [pos_v[s, sl] for sl in sls]
                vas = [buf[ra, sl] for sl in sls]
                vbs = [buf[rb, sl] for sl in sls]
                for sl, pv, va in zip(sls, pvs, vas):
                    buf[ra, sl] = (va + pv) * mak
                for sl, pv, vb in zip(sls, pvs, vbs):
                    buf[rb, sl] = (vb + pv) * mbk

        @plsc.parallel_loop(0, NFULL[half] * 2)
        def _sgrp(i):
            process8(i * 8)

        if NTAIL[half]:
            process16(NFULL[half] * LANES, NTAIL[half])

        for d in odescs(p, half, sp):
            d.start()

        def drain_prev():
            pp, ph = (p - 1, 1) if half == 0 else (p, 0)
            for d in odescs(pp, ph, spn):
                d.wait()

        if half == 0:
            @pl.when(c >= 1)
            def _():
                drain_prev()
        else:
            drain_prev()

        @pl.when(p < PAIRS - 1)
        def _():
            for d in gdescs(p + 1, half, spn):
                d.start()

    for d in gdescs(0, 0, 0):
        d.start()
    for d in gdescs(0, 1, 1):
        d.start()

    def pair_body(p, _):
        step(p, 0)
        step(p, 1)
        return 0

    lax.fori_loop(0, PAIRS, pair_body, 0)
    for d in odescs(PAIRS - 1, 1, (2 * PAIRS - 1) % NBUF):
        d.wait()


@jax.jit
def _embed(idx_flat, token_table, position_table):
    mesh = plsc.VectorSubcoreMesh(core_axis_name="c", subcore_axis_name="s")
    kern = functools.partial(
        pl.kernel,
        out_type=jax.ShapeDtypeStruct((BATCH * SEQ_LEN, EMBED_DIM),
                                      jnp.float32),
        mesh=mesh,
        scratch_types=[
            pltpu.VMEM((IDX_PER_W + LANES,), jnp.int32),
            pltpu.VMEM((SEQ_LEN, EMBED_DIM), jnp.float32),
            pltpu.VMEM((NBUF * 2 * CH, EMBED_DIM), jnp.float32),
            pltpu.SemaphoreType.DMA,
            pltpu.SemaphoreType.DMA,
        ],
    )(_embed_kernel)
    return kern(idx_flat, token_table, position_table)


def kernel(inputs, token_table, position_table):
    idx_flat = inputs.astype(jnp.int32).reshape(-1)
    out = _embed(idx_flat, token_table, position_table)
    return out.reshape(BATCH, SEQ_LEN, EMBED_DIM)
